# Initial kernel scaffold; baseline (speedup 1.0000x reference)
import jax
import jax.numpy as jnp
from jax.experimental import pallas as pl


def kernel(sem_ids, encoded_features):
    raise NotImplementedError("write your pallas kernel here")


# speedup vs baseline: 230.4192x; 230.4192x over previous
import jax
import jax.numpy as jnp
from jax.experimental import pallas as pl
from jax.experimental.pallas import tpu as pltpu

MARGIN = 0.5
WEIGHT = 1.0

_BI = 256


def _loss_body(sem_ref, semt_ref, feat_ref, out_ref,
               fn_ref, kcol_ref, krow_ref, tot_ref, cnt_ref):
    i = pl.program_id(0)
    nsteps = pl.num_programs(0)
    b = feat_ref.shape[0]

    @pl.when(i == 0)
    def _init():
        f = feat_ref[...]
        ss = jnp.sum(f * f, axis=1, keepdims=True)
        norm = jnp.maximum(jnp.sqrt(ss), 1e-12)
        fn_ref[...] = f / norm
        s = sem_ref[...]
        kcol_ref[...] = (
            ((s[:, 0:1] * 8 + s[:, 1:2]) * 8 + s[:, 2:3]) * 8 + s[:, 3:4]
        )
        st = semt_ref[...]
        krow_ref[...] = (
            ((st[0:1, :] * 8 + st[1:2, :]) * 8 + st[2:3, :]) * 8 + st[3:4, :]
        )
        tot_ref[0, 0] = 0.0
        cnt_ref[0, 0] = 0.0

    fi = fn_ref[pl.ds(i * _BI, _BI), :]
    g = jax.lax.dot_general(
        fi, fn_ref[...], (((1,), (1,)), ((), ())),
        preferred_element_type=jnp.float32,
    )
    ki = kcol_ref[pl.ds(i * _BI, _BI), :]
    kj = krow_ref[...]
    rows = i * _BI + jax.lax.broadcasted_iota(jnp.int32, (_BI, b), 0)
    cols = jax.lax.broadcasted_iota(jnp.int32, (_BI, b), 1)
    mask = (ki == kj) & (rows < cols)
    hinge = jnp.maximum(g - MARGIN, 0.0)
    tot_ref[0, 0] += jnp.sum(jnp.where(mask, hinge, 0.0))
    cnt_ref[0, 0] += jnp.sum(mask.astype(jnp.float32))

    @pl.when(i == nsteps - 1)
    def _fin():
        cnt = cnt_ref[0, 0]
        tot = tot_ref[0, 0]
        mean = WEIGHT * tot / jnp.maximum(cnt, 1.0)
        out_ref[0, 0] = jnp.where(cnt > 0.0, mean, 0.0)


@jax.jit
def kernel(sem_ids, encoded_features):
    b, d = encoded_features.shape
    nid = sem_ids.shape[1]
    grid = b // _BI
    out = pl.pallas_call(
        _loss_body,
        grid=(grid,),
        in_specs=[
            pl.BlockSpec((b, nid), lambda i: (0, 0)),
            pl.BlockSpec((nid, b), lambda i: (0, 0)),
            pl.BlockSpec((b, d), lambda i: (0, 0)),
        ],
        out_specs=pl.BlockSpec(memory_space=pltpu.SMEM),
        out_shape=jax.ShapeDtypeStruct((1, 1), jnp.float32),
        scratch_shapes=[
            pltpu.VMEM((b, d), jnp.float32),
            pltpu.VMEM((b, 1), jnp.int32),
            pltpu.VMEM((1, b), jnp.int32),
            pltpu.SMEM((1, 1), jnp.float32),
            pltpu.SMEM((1, 1), jnp.float32),
        ],
    )(sem_ids, sem_ids.T, encoded_features)
    return out[0, 0]


# speedup vs baseline: 251.0754x; 1.0896x over previous
import jax
import jax.numpy as jnp
from jax.experimental import pallas as pl
from jax.experimental.pallas import tpu as pltpu

MARGIN = 0.5
WEIGHT = 1.0

_BL = 512


def _loss_body(sem_ref, semt_ref, feat_ref, out_ref,
               fn_ref, kcol_ref, krow_ref, tot_ref, cnt_ref):
    i = pl.program_id(0)
    j = pl.program_id(1)
    ni = pl.num_programs(0)
    nj = pl.num_programs(1)

    @pl.when((i == 0) & (j == 0))
    def _init():
        f = feat_ref[...]
        ss = jnp.sum(f * f, axis=1, keepdims=True)
        norm = jnp.maximum(jnp.sqrt(ss), 1e-12)
        fn_ref[...] = f / norm
        s = sem_ref[...]
        kcol_ref[...] = (
            ((s[:, 0:1] * 8 + s[:, 1:2]) * 8 + s[:, 2:3]) * 8 + s[:, 3:4]
        )
        st = semt_ref[...]
        krow_ref[...] = (
            ((st[0:1, :] * 8 + st[1:2, :]) * 8 + st[2:3, :]) * 8 + st[3:4, :]
        )
        tot_ref[0, 0] = 0.0
        cnt_ref[0, 0] = 0.0

    @pl.when(j >= i)
    def _compute():
        fi = fn_ref[pl.ds(i * _BL, _BL), :]
        fj = fn_ref[pl.ds(j * _BL, _BL), :]
        g = jax.lax.dot_general(
            fi, fj, (((1,), (1,)), ((), ())),
            preferred_element_type=jnp.float32,
        )
        ki = kcol_ref[pl.ds(i * _BL, _BL), :]
        kj = krow_ref[:, pl.ds(j * _BL, _BL)]
        d = (jax.lax.broadcasted_iota(jnp.int32, (_BL, _BL), 0)
             - jax.lax.broadcasted_iota(jnp.int32, (_BL, _BL), 1))
        mask = (ki == kj) & (d < (j - i) * _BL)
        hinge = jnp.maximum(g - MARGIN, 0.0)
        tot_ref[0, 0] += jnp.sum(jnp.where(mask, hinge, 0.0))
        cnt_ref[0, 0] += jnp.sum(mask.astype(jnp.float32))

    @pl.when((i == ni - 1) & (j == nj - 1))
    def _fin():
        cnt = cnt_ref[0, 0]
        tot = tot_ref[0, 0]
        mean = WEIGHT * tot / jnp.maximum(cnt, 1.0)
        out_ref[0, 0] = jnp.where(cnt > 0.0, mean, 0.0)


@jax.jit
def kernel(sem_ids, encoded_features):
    b, d = encoded_features.shape
    nid = sem_ids.shape[1]
    grid = b // _BL
    out = pl.pallas_call(
        _loss_body,
        grid=(grid, grid),
        in_specs=[
            pl.BlockSpec((b, nid), lambda i, j: (0, 0)),
            pl.BlockSpec((nid, b), lambda i, j: (0, 0)),
            pl.BlockSpec((b, d), lambda i, j: (0, 0)),
        ],
        out_specs=pl.BlockSpec(memory_space=pltpu.SMEM),
        out_shape=jax.ShapeDtypeStruct((1, 1), jnp.float32),
        scratch_shapes=[
            pltpu.VMEM((b, d), jnp.float32),
            pltpu.VMEM((b, 1), jnp.int32),
            pltpu.VMEM((1, b), jnp.int32),
            pltpu.SMEM((1, 1), jnp.float32),
            pltpu.SMEM((1, 1), jnp.float32),
        ],
    )(sem_ids, sem_ids.T, encoded_features)
    return out[0, 0]
